# Initial kernel scaffold; baseline (speedup 1.0000x reference)
import jax
import jax.numpy as jnp
from jax.experimental import pallas as pl


def kernel(x_paper, x_author, edge_index_cites, edge_index_writes, edge_index_rev_writes, params):
    raise NotImplementedError("write your pallas kernel here")


# speedup vs baseline: 9.1481x; 9.1481x over previous
import functools

import jax
import jax.numpy as jnp
import numpy as np
from jax import lax
from jax.experimental import pallas as pl
from jax.experimental.pallas import tpu as pltpu
from jax.experimental.pallas import tpu_sc as plsc

H = 8
D = 16
HID = 128
OUT = 64
NCORE = 2
NSUB = 16
NW = NCORE * NSUB
B = 64

N_PAPER = 50000
N_AUTHOR = 20000


def _proj_body(x_ref, w_ref, b_ref, *o_refs, widths, act):
    y = jnp.dot(x_ref[...], w_ref[...], preferred_element_type=jnp.float32)
    y = y + b_ref[...]
    if act == "relu":
        y = jnp.maximum(y, 0.0)
    off = 0
    for r, w in zip(o_refs, widths):
        r[...] = y[:, off:off + w]
        off += w


def _proj(x, wwide, bwide, widths, act=None, bn=400):
    n = x.shape[0]
    wt = wwide.shape[1]
    grid = (n // bn,)
    return pl.pallas_call(
        functools.partial(_proj_body, widths=tuple(widths), act=act),
        grid=grid,
        in_specs=[
            pl.BlockSpec((bn, 128), lambda i: (i, 0)),
            pl.BlockSpec((128, wt), lambda i: (0, 0)),
            pl.BlockSpec((1, wt), lambda i: (0, 0)),
        ],
        out_specs=[pl.BlockSpec((bn, w), lambda i: (i, 0)) for w in widths],
        out_shape=[jax.ShapeDtypeStruct((n, w), jnp.float32) for w in widths],
    )(x, wwide, bwide.reshape(1, wt))


def _combine_body(*refs, n_et, bn):
    num_refs = [refs[2 * e] for e in range(n_et)]
    den_refs = [refs[2 * e + 1] for e in range(n_et)]
    e8_ref, x_ref, aw_ref, ab_ref, bl_ref = refs[2 * n_et:2 * n_et + 5]
    o_ref = refs[2 * n_et + 5]
    acc = jnp.zeros((bn, HID), jnp.float32)
    for e in range(n_et):
        nr = num_refs[e][...]
        dr = den_refs[e][...]
        ntot = nr[0] + nr[1]
        dtot = dr[0] + dr[1]
        ncat = jnp.concatenate([ntot[p] for p in range(H)], axis=1)
        dx = jnp.dot(dtot, e8_ref[...], preferred_element_type=jnp.float32)
        acc = acc + ncat / (dx + 1e-16)
    g = jax.nn.gelu(acc, approximate=True)
    o = jnp.dot(g, aw_ref[...], preferred_element_type=jnp.float32) + ab_ref[...]
    bl = bl_ref[...]
    o_ref[...] = bl * o + (1.0 - bl) * x_ref[...]


def _combine(num_den_list, x_prev, aw, ab, blend_vec, bn=400):
    n = x_prev.shape[0]
    n_et = len(num_den_list)
    e8np = np.zeros((16, 128), np.float32)
    for h in range(8):
        e8np[h, 16 * h:16 * (h + 1)] = 1.0
    e8 = jnp.asarray(e8np)
    args = []
    in_specs = []
    def _nmap(i):
        return (0, 0, jnp.where(i >= RNG // bn, i + 1, i), 0)

    def _dmap(i):
        return (0, jnp.where(i >= RNG // bn, i + 1, i), 0)

    for (num, den) in num_den_list:
        args += [num, den]
        in_specs += [
            pl.BlockSpec((2, 8, bn, 16), _nmap),
            pl.BlockSpec((2, bn, 16), _dmap),
        ]
    args += [e8, x_prev, aw, ab.reshape(1, HID), blend_vec]
    in_specs += [
        pl.BlockSpec((16, 128), lambda i: (0, 0)),
        pl.BlockSpec((bn, 128), lambda i: (i, 0)),
        pl.BlockSpec((128, 128), lambda i: (0, 0)),
        pl.BlockSpec((1, 128), lambda i: (0, 0)),
        pl.BlockSpec((1, 128), lambda i: (0, 0)),
    ]
    return pl.pallas_call(
        functools.partial(_combine_body, n_et=n_et, bn=bn),
        grid=(n // bn,),
        in_specs=in_specs,
        out_specs=pl.BlockSpec((bn, 128), lambda i: (i, 0)),
        out_shape=jax.ShapeDtypeStruct((n, 128), jnp.float32),
    )(*args)


NPR = 26000
RNG = 25600


def _sc_body(meta_h, qtab, ktab, vt0, vt1, vt2, vt3, vt4, vt5, vt6, vt7,
             src_h, dstg_h, dsts_h,
             num_h, den_h,
             meta_v, src_v, dstg_v, dsts_v, didx_v, qrows, krows, vrows,
             msgv, denrows, exv, zv, num_sh, den_sh, sem, *, NB):
    c = lax.axis_index("c")
    s = lax.axis_index("s")
    wid = s * NCORE + c
    vts = [vt0, vt1, vt2, vt3, vt4, vt5, vt6, vt7]

    pltpu.sync_copy(meta_h, meta_v)
    nb_used = meta_v[pl.ds(0, 16)][0]

    z16 = jnp.zeros((16,), jnp.float32)

    def _zrow(r, _):
        zv[r, pl.ds(0, 16)] = z16
        return 0
    lax.fori_loop(0, 128, _zrow, 0)

    rows_per_tile = NPR // NSUB
    nfull = rows_per_tile // 128
    nrem = rows_per_tile - nfull * 128
    r0 = s * rows_per_tile

    def _zero_rows(buf):
        def zb(i, _):
            pltpu.sync_copy(zv, buf.at[pl.ds(r0 + i * 128, 128)])
            return 0
        lax.fori_loop(0, nfull, zb, 0)
        pltpu.sync_copy(zv.at[pl.ds(0, nrem)],
                        buf.at[pl.ds(r0 + nfull * 128, nrem)])

    def _out_rows(buf, dst, roff):
        def ob(i, _):
            off = r0 + i * 128
            pltpu.sync_copy(buf.at[pl.ds(off, 128)],
                            dst.at[pl.ds(roff + off, 128)])
            return 0
        lax.fori_loop(0, nfull, ob, 0)
        off = r0 + nfull * 128
        pltpu.sync_copy(buf.at[pl.ds(off, nrem)],
                        dst.at[pl.ds(roff + off, nrem)])

    _zero_rows(num_sh)
    _zero_rows(den_sh)

    def ex_batch(j, _):
        pltpu.sync_copy(dstg_h.at[wid, j], dstg_v)
        pltpu.sync_copy(src_h.at[wid, j], src_v)
        pltpu.async_copy(qtab.at[dstg_v], qrows, sem).wait()
        pltpu.async_copy(ktab.at[src_v], krows, sem).wait()

        def grp(g, _):
            ev = g * 16 + lax.iota(jnp.int32, 16)
            for h in range(H):
                acc = jnp.zeros((16,), jnp.float32)
                for d in range(D):
                    col = jnp.full((16,), h * D + d, jnp.int32)
                    acc = acc + (plsc.load_gather(qrows, [ev, col]) *
                                 plsc.load_gather(krows, [ev, col]))
                exv[h, pl.ds(j * B + g * 16, 16)] = jnp.exp(acc)
            return 0
        lax.fori_loop(0, B // 16, grp, 0)
        return 0
    lax.fori_loop(0, nb_used, ex_batch, 0)

    for r in range(2):
        base = r * RNG

        def db(j, _):
            pltpu.sync_copy(dsts_h.at[wid, j], dsts_v)

            def dg(g, _):
                dv = dsts_v[pl.ds(g * 16, 16)]
                ld = dv - base
                ok = (ld >= 0) & (ld < RNG)
                didx_v[j, pl.ds(g * 16, 16)] = jnp.where(ok, ld, RNG)
                return 0
            lax.fori_loop(0, B // 16, dg, 0)
            return 0
        lax.fori_loop(0, nb_used, db, 0)

        plsc.subcore_barrier()

        def den_batch(j, _):
            def dgrp(g, _):
                hmod = lax.iota(jnp.int32, 16) & 7
                for i in range(16):
                    e = g * 16 + i
                    ge = jnp.full((16,), j * B + e, jnp.int32)
                    denrows[e, pl.ds(0, 16)] = plsc.load_gather(exv, [hmod, ge])
                return 0
            lax.fori_loop(0, B // 16, dgrp, 0)
            pltpu.sync_copy(denrows, den_sh.at[didx_v.at[j]], add=True)
            return 0
        lax.fori_loop(0, nb_used, den_batch, 0)

        for p in range(H):
            if p > 0:
                plsc.subcore_barrier()

            def cp_batch(j, _):
                pltpu.sync_copy(src_h.at[wid, j], src_v)
                pltpu.async_copy(vts[p].at[src_v], vrows, sem).wait()

                def grp(g, _):
                    e0 = g * 16
                    exvec = exv[p, pl.ds(j * B + e0, 16)]
                    for i in range(16):
                        msgv[e0 + i, pl.ds(0, 16)] = (
                            vrows[e0 + i, pl.ds(0, 16)] * exvec[i])
                    return 0
                lax.fori_loop(0, B // 16, grp, 0)
                pltpu.sync_copy(msgv, num_sh.at[didx_v.at[j]], add=True)
                return 0
            lax.fori_loop(0, nb_used, cp_batch, 0)

            plsc.subcore_barrier()

            if p == 0:
                _out_rows(den_sh, den_h.at[c], r * NPR)
                if r == 0:
                    _zero_rows(den_sh)
            _out_rows(num_sh, num_h.at[c, p], r * NPR)
            if not (r == 1 and p == H - 1):
                _zero_rows(num_sh)


OUT_ROWS = 2 * NPR


@functools.lru_cache(maxsize=None)
def _make_sc(NB):
    ew = NB * B
    mesh = plsc.VectorSubcoreMesh(core_axis_name="c", subcore_axis_name="s")
    return functools.partial(
        pl.kernel,
        mesh=mesh,
        compiler_params=pltpu.CompilerParams(
            needs_layout_passes=False, use_tc_tiling_on_sc=False),
        out_type=[
            jax.ShapeDtypeStruct((NCORE, H, OUT_ROWS, 16), jnp.float32),
            jax.ShapeDtypeStruct((NCORE, OUT_ROWS, 16), jnp.float32),
        ],
        scratch_types=[
            pltpu.VMEM((16,), jnp.int32),
            pltpu.VMEM((B,), jnp.int32),
            pltpu.VMEM((B,), jnp.int32),
            pltpu.VMEM((B,), jnp.int32),
            pltpu.VMEM((NB, B), jnp.int32),
            pltpu.VMEM((B, 128), jnp.float32),
            pltpu.VMEM((B, 128), jnp.float32),
            pltpu.VMEM((B, 16), jnp.float32),
            pltpu.VMEM((B, 16), jnp.float32),
            pltpu.VMEM((B, 16), jnp.float32),
            pltpu.VMEM((H, ew), jnp.float32),
            pltpu.VMEM((128, 16), jnp.float32),
            pltpu.VMEM_SHARED((NPR, 16), jnp.float32),
            pltpu.VMEM_SHARED((NPR, 16), jnp.float32),
            pltpu.SemaphoreType.DMA,
        ],
    )(functools.partial(_sc_body, NB=NB))


NB_SC = 98
GARBAGE_ROW = 50000


def _prep_edges(ei, nb_real):
    e = ei.shape[1]
    ewr = nb_real * B
    epad = NW * ewr
    src = jnp.concatenate([ei[0], jnp.zeros((epad - e,), jnp.int32)])
    dstg = jnp.concatenate([ei[1], jnp.zeros((epad - e,), jnp.int32)])
    dsts = jnp.concatenate([ei[1], jnp.full((epad - e,), GARBAGE_ROW,
                                            jnp.int32)])

    def expand(a, fill):
        a2 = a.reshape(NW, ewr)
        pad = jnp.full((NW, (NB_SC - nb_real) * B), fill, jnp.int32)
        return jnp.concatenate([a2, pad], axis=1).reshape(NW, NB_SC, B)

    meta = jnp.full((16,), nb_real, jnp.int32)
    return (meta, expand(src, 0), expand(dstg, 0),
            expand(dsts, GARBAGE_ROW))


def _block_diag(a):
    out = jnp.zeros((H * D, H * D), jnp.float32)
    for h in range(H):
        out = out.at[h * D:(h + 1) * D, h * D:(h + 1) * D].set(a[h])
    return out


def _fold(layer, et, src_t):
    pe = layer["edge"][et]
    p = layer["node"][src_t]
    scale = jnp.repeat(pe["p_rel"] / jnp.sqrt(jnp.float32(D)), D)
    bda = _block_diag(pe["a_rel"])
    bdm = _block_diag(pe["m_rel"])
    kw = (p["k_w"] @ bda) * scale[None, :]
    kb = (p["k_b"] @ bda) * scale
    vw = p["v_w"] @ bdm
    vb = p["v_b"] @ bdm
    return kw, kb, vw, vb


def kernel(x_paper, x_author, edge_index_cites, edge_index_writes,
           edge_index_rev_writes, params):
    ec = _prep_edges(edge_index_cites, 98)
    ew_ = _prep_edges(edge_index_writes, 49)
    er = _prep_edges(edge_index_rev_writes, 49)

    xp = _proj(x_paper, params["lin_in"]["paper"]["w"],
               params["lin_in"]["paper"]["b"], [HID], act="relu")[0]
    xa = _proj(x_author, params["lin_in"]["author"]["w"],
               params["lin_in"]["author"]["b"], [HID], act="relu")[0]

    for layer in params["layers"]:
        pp = layer["node"]["paper"]
        pa = layer["node"]["author"]
        kwc, kbc, vwc, vbc = _fold(layer, "cites", "paper")
        kwr, kbr, vwr, vbr = _fold(layer, "rev_writes", "paper")
        kww, kbw, vww, vbw = _fold(layer, "writes", "author")

        wp = jnp.concatenate([pp["q_w"], kwc, kwr, vwc, vwr], axis=1)
        bp = jnp.concatenate([pp["q_b"], kbc, kbr, vbc, vbr])
        widths_p = [128, 128, 128] + [16] * 16
        outs_p = _proj(xp, wp, bp, widths_p)
        qp, kc, kr = outs_p[0], outs_p[1], outs_p[2]
        vc = outs_p[3:11]
        vr = outs_p[11:19]

        wa = jnp.concatenate([pa["q_w"], kww, vww], axis=1)
        ba = jnp.concatenate([pa["q_b"], kbw, vbw])
        xa_pad = jnp.concatenate(
            [xa, jnp.zeros((N_PAPER - N_AUTHOR, HID), jnp.float32)])
        outs_a = _proj(xa_pad, wa, ba, [128, 128] + [16] * 8)
        qa, kw_ = outs_a[0], outs_a[1]
        vw8 = outs_a[2:10]

        sck = _make_sc(NB_SC)
        numc, denc = sck(ec[0], qp, kc, *vc, *ec[1:])
        numw, denw = sck(ew_[0], qp, kw_, *vw8, *ew_[1:])
        numr, denr = sck(er[0], qa, kr, *vr, *er[1:])

        blp = jnp.broadcast_to(jax.nn.sigmoid(pp["skip"]), (1, HID))
        bla = jnp.broadcast_to(jax.nn.sigmoid(pa["skip"]), (1, HID))
        xp_new = _combine(
            [(numc, denc), (numw, denw)],
            xp, pp["a_w"], pp["a_b"], blp)
        xa_new = _combine(
            [(numr, denr)],
            xa, pa["a_w"], pa["a_b"], bla)
        xp, xa = xp_new, xa_new

    out = _proj(xp, params["lin_out"]["w"], params["lin_out"]["b"], [OUT])[0]
    return (out, xp, xa)


# speedup vs baseline: 9.4274x; 1.0305x over previous
import functools

import jax
import jax.numpy as jnp
import numpy as np
from jax import lax
from jax.experimental import pallas as pl
from jax.experimental.pallas import tpu as pltpu
from jax.experimental.pallas import tpu_sc as plsc

H = 8
D = 16
HID = 128
OUT = 64
NCORE = 2
NSUB = 16
NW = NCORE * NSUB
B = 64

N_PAPER = 50000
N_AUTHOR = 20000


def _proj_body(x_ref, w_ref, b_ref, *o_refs, widths, act):
    y = jnp.dot(x_ref[...], w_ref[...], preferred_element_type=jnp.float32)
    y = y + b_ref[...]
    if act == "relu":
        y = jnp.maximum(y, 0.0)
    off = 0
    for r, w in zip(o_refs, widths):
        r[...] = y[:, off:off + w]
        off += w


def _proj(x, wwide, bwide, widths, act=None, bn=400):
    n = x.shape[0]
    wt = wwide.shape[1]
    grid = (n // bn,)
    return pl.pallas_call(
        functools.partial(_proj_body, widths=tuple(widths), act=act),
        grid=grid,
        in_specs=[
            pl.BlockSpec((bn, 128), lambda i: (i, 0)),
            pl.BlockSpec((128, wt), lambda i: (0, 0)),
            pl.BlockSpec((1, wt), lambda i: (0, 0)),
        ],
        out_specs=[pl.BlockSpec((bn, w), lambda i: (i, 0)) for w in widths],
        out_shape=[jax.ShapeDtypeStruct((n, w), jnp.float32) for w in widths],
    )(x, wwide, bwide.reshape(1, wt))


def _combine_body(*refs, n_et, bn):
    num_refs = [refs[2 * e] for e in range(n_et)]
    den_refs = [refs[2 * e + 1] for e in range(n_et)]
    e8_ref, x_ref, aw_ref, ab_ref, bl_ref = refs[2 * n_et:2 * n_et + 5]
    o_ref = refs[2 * n_et + 5]
    acc = jnp.zeros((bn, HID), jnp.float32)
    for e in range(n_et):
        nr = num_refs[e][...]
        dr = den_refs[e][...]
        ntot = nr[0] + nr[1]
        dtot = dr[0] + dr[1]
        ncat = jnp.concatenate([ntot[p] for p in range(H)], axis=1)
        dx = jnp.dot(dtot, e8_ref[...], preferred_element_type=jnp.float32)
        acc = acc + ncat / (dx + 1e-16)
    g = jax.nn.gelu(acc, approximate=True)
    o = jnp.dot(g, aw_ref[...], preferred_element_type=jnp.float32) + ab_ref[...]
    bl = bl_ref[...]
    o_ref[...] = bl * o + (1.0 - bl) * x_ref[...]


def _combine(num_den_list, x_prev, aw, ab, blend_vec, bn=400):
    n = x_prev.shape[0]
    n_et = len(num_den_list)
    e8np = np.zeros((16, 128), np.float32)
    for h in range(8):
        e8np[h, 16 * h:16 * (h + 1)] = 1.0
    e8 = jnp.asarray(e8np)
    args = []
    in_specs = []
    def _nmap(i):
        return (0, 0, jnp.where(i >= RNG // bn, i + 1, i), 0)

    def _dmap(i):
        return (0, jnp.where(i >= RNG // bn, i + 1, i), 0)

    for (num, den) in num_den_list:
        args += [num, den]
        in_specs += [
            pl.BlockSpec((2, 8, bn, 16), _nmap),
            pl.BlockSpec((2, bn, 16), _dmap),
        ]
    args += [e8, x_prev, aw, ab.reshape(1, HID), blend_vec]
    in_specs += [
        pl.BlockSpec((16, 128), lambda i: (0, 0)),
        pl.BlockSpec((bn, 128), lambda i: (i, 0)),
        pl.BlockSpec((128, 128), lambda i: (0, 0)),
        pl.BlockSpec((1, 128), lambda i: (0, 0)),
        pl.BlockSpec((1, 128), lambda i: (0, 0)),
    ]
    return pl.pallas_call(
        functools.partial(_combine_body, n_et=n_et, bn=bn),
        grid=(n // bn,),
        in_specs=in_specs,
        out_specs=pl.BlockSpec((bn, 128), lambda i: (i, 0)),
        out_shape=jax.ShapeDtypeStruct((n, 128), jnp.float32),
    )(*args)


NPR = 26000
RNG = 25600


def _sc_body(meta_h, qtab, ktab, vt0, vt1, vt2, vt3, vt4, vt5, vt6, vt7,
             src_h, dstg_h, dsts_h,
             num_h, den_h,
             meta_v, src_v, dstg_v, dsts_v, didx_v, qrows, krows, vrows,
             msgv, denrows, exv, zv, num_sh, den_sh, sem, *, NB):
    c = lax.axis_index("c")
    s = lax.axis_index("s")
    wid = s * NCORE + c
    vts = [vt0, vt1, vt2, vt3, vt4, vt5, vt6, vt7]

    pltpu.sync_copy(meta_h.at[wid], meta_v)
    mvec = meta_v[pl.ds(0, 16)]
    nb_used = mvec[0]
    b0_hi = mvec[1]
    b1_lo = mvec[2]

    z16 = jnp.zeros((16,), jnp.float32)

    def _zrow(r, _):
        zv[r, pl.ds(0, 16)] = z16
        return 0
    lax.fori_loop(0, 128, _zrow, 0)

    rows_per_tile = NPR // NSUB
    nfull = rows_per_tile // 128
    nrem = rows_per_tile - nfull * 128
    r0 = s * rows_per_tile

    def _zero_rows(buf):
        def zb(i, _):
            pltpu.sync_copy(zv, buf.at[pl.ds(r0 + i * 128, 128)])
            return 0
        lax.fori_loop(0, nfull, zb, 0)
        pltpu.sync_copy(zv.at[pl.ds(0, nrem)],
                        buf.at[pl.ds(r0 + nfull * 128, nrem)])

    def _out_rows(buf, dst, roff):
        def ob(i, _):
            off = r0 + i * 128
            pltpu.sync_copy(buf.at[pl.ds(off, 128)],
                            dst.at[pl.ds(roff + off, 128)])
            return 0
        lax.fori_loop(0, nfull, ob, 0)
        off = r0 + nfull * 128
        pltpu.sync_copy(buf.at[pl.ds(off, nrem)],
                        dst.at[pl.ds(roff + off, nrem)])

    _zero_rows(num_sh)
    _zero_rows(den_sh)

    def ex_batch(j, _):
        pltpu.sync_copy(dstg_h.at[wid, j], dstg_v)
        pltpu.sync_copy(src_h.at[wid, j], src_v)
        pltpu.async_copy(qtab.at[dstg_v], qrows, sem).wait()
        pltpu.async_copy(ktab.at[src_v], krows, sem).wait()

        def grp(g, _):
            ev = g * 16 + lax.iota(jnp.int32, 16)
            for h in range(H):
                acc = jnp.zeros((16,), jnp.float32)
                for d in range(D):
                    col = jnp.full((16,), h * D + d, jnp.int32)
                    acc = acc + (plsc.load_gather(qrows, [ev, col]) *
                                 plsc.load_gather(krows, [ev, col]))
                exv[h, pl.ds(j * B + g * 16, 16)] = jnp.exp(acc)
            return 0
        lax.fori_loop(0, B // 16, grp, 0)
        return 0
    lax.fori_loop(0, nb_used, ex_batch, 0)

    for r in range(2):
        base = r * RNG
        j_lo = 0 if r == 0 else b1_lo
        j_hi = b0_hi if r == 0 else nb_used

        def db(j, _):
            pltpu.sync_copy(dsts_h.at[wid, j], dsts_v)

            def dg(g, _):
                dv = dsts_v[pl.ds(g * 16, 16)]
                ld = dv - base
                ok = (ld >= 0) & (ld < RNG)
                didx_v[j, pl.ds(g * 16, 16)] = jnp.where(ok, ld, RNG)
                return 0
            lax.fori_loop(0, B // 16, dg, 0)
            return 0
        lax.fori_loop(j_lo, j_hi, db, 0)

        plsc.subcore_barrier()

        def den_batch(j, _):
            def dgrp(g, _):
                hmod = lax.iota(jnp.int32, 16) & 7
                for i in range(16):
                    e = g * 16 + i
                    ge = jnp.full((16,), j * B + e, jnp.int32)
                    denrows[e, pl.ds(0, 16)] = plsc.load_gather(exv, [hmod, ge])
                return 0
            lax.fori_loop(0, B // 16, dgrp, 0)
            pltpu.sync_copy(denrows, den_sh.at[didx_v.at[j]], add=True)
            return 0
        lax.fori_loop(j_lo, j_hi, den_batch, 0)

        for p in range(H):
            if p > 0:
                plsc.subcore_barrier()

            def cp_batch(j, _):
                pltpu.sync_copy(src_h.at[wid, j], src_v)
                pltpu.async_copy(vts[p].at[src_v], vrows, sem).wait()

                def grp(g, _):
                    e0 = g * 16
                    exvec = exv[p, pl.ds(j * B + e0, 16)]
                    for i in range(16):
                        msgv[e0 + i, pl.ds(0, 16)] = (
                            vrows[e0 + i, pl.ds(0, 16)] * exvec[i])
                    return 0
                lax.fori_loop(0, B // 16, grp, 0)
                pltpu.sync_copy(msgv, num_sh.at[didx_v.at[j]], add=True)
                return 0
            lax.fori_loop(j_lo, j_hi, cp_batch, 0)

            plsc.subcore_barrier()

            if p == 0:
                _out_rows(den_sh, den_h.at[c], r * NPR)
                if r == 0:
                    _zero_rows(den_sh)
            _out_rows(num_sh, num_h.at[c, p], r * NPR)
            if not (r == 1 and p == H - 1):
                _zero_rows(num_sh)


OUT_ROWS = 2 * NPR


@functools.lru_cache(maxsize=None)
def _make_sc(NB):
    ew = NB * B
    mesh = plsc.VectorSubcoreMesh(core_axis_name="c", subcore_axis_name="s")
    return functools.partial(
        pl.kernel,
        mesh=mesh,
        compiler_params=pltpu.CompilerParams(
            needs_layout_passes=False, use_tc_tiling_on_sc=False),
        out_type=[
            jax.ShapeDtypeStruct((NCORE, H, OUT_ROWS, 16), jnp.float32),
            jax.ShapeDtypeStruct((NCORE, OUT_ROWS, 16), jnp.float32),
        ],
        scratch_types=[
            pltpu.VMEM((16,), jnp.int32),
            pltpu.VMEM((B,), jnp.int32),
            pltpu.VMEM((B,), jnp.int32),
            pltpu.VMEM((B,), jnp.int32),
            pltpu.VMEM((NB, B), jnp.int32),
            pltpu.VMEM((B, 128), jnp.float32),
            pltpu.VMEM((B, 128), jnp.float32),
            pltpu.VMEM((B, 16), jnp.float32),
            pltpu.VMEM((B, 16), jnp.float32),
            pltpu.VMEM((B, 16), jnp.float32),
            pltpu.VMEM((H, ew), jnp.float32),
            pltpu.VMEM((128, 16), jnp.float32),
            pltpu.VMEM_SHARED((NPR, 16), jnp.float32),
            pltpu.VMEM_SHARED((NPR, 16), jnp.float32),
            pltpu.SemaphoreType.DMA,
        ],
    )(functools.partial(_sc_body, NB=NB))


NB_SC = 98
GARBAGE_ROW = 50000


def _prep_edges(ei, nb_real):
    e = ei.shape[1]
    order = jnp.argsort(ei[1] >= RNG, stable=True)
    s_srt = ei[0][order]
    d_srt = ei[1][order]
    ewr = nb_real * B
    epad = NW * ewr
    src = jnp.concatenate([s_srt, jnp.zeros((epad - e,), jnp.int32)])
    dstg = jnp.concatenate([d_srt, jnp.zeros((epad - e,), jnp.int32)])
    dsts = jnp.concatenate([d_srt, jnp.full((epad - e,), GARBAGE_ROW,
                                            jnp.int32)])

    def expand(a, fill):
        a2 = a.reshape(NW, ewr)
        pad = jnp.full((NW, (NB_SC - nb_real) * B), fill, jnp.int32)
        return jnp.concatenate([a2, pad], axis=1).reshape(NW, NB_SC, B)

    count0 = jnp.sum(ei[1] < RNG).astype(jnp.int32)
    c0w = jnp.clip(count0 - jnp.arange(NW, dtype=jnp.int32) * ewr, 0, ewr)
    b0_hi = (c0w + B - 1) // B
    b1_lo = c0w // B
    meta = jnp.stack(
        [jnp.full((NW,), nb_real, jnp.int32), b0_hi, b1_lo]
        + [jnp.zeros((NW,), jnp.int32)] * 13, axis=1)
    return (meta, expand(src, 0), expand(dstg, 0),
            expand(dsts, GARBAGE_ROW))


def _block_diag(a):
    out = jnp.zeros((H * D, H * D), jnp.float32)
    for h in range(H):
        out = out.at[h * D:(h + 1) * D, h * D:(h + 1) * D].set(a[h])
    return out


def _fold(layer, et, src_t):
    pe = layer["edge"][et]
    p = layer["node"][src_t]
    scale = jnp.repeat(pe["p_rel"] / jnp.sqrt(jnp.float32(D)), D)
    bda = _block_diag(pe["a_rel"])
    bdm = _block_diag(pe["m_rel"])
    kw = (p["k_w"] @ bda) * scale[None, :]
    kb = (p["k_b"] @ bda) * scale
    vw = p["v_w"] @ bdm
    vb = p["v_b"] @ bdm
    return kw, kb, vw, vb


def kernel(x_paper, x_author, edge_index_cites, edge_index_writes,
           edge_index_rev_writes, params):
    ec = _prep_edges(edge_index_cites, 98)
    ew_ = _prep_edges(edge_index_writes, 49)
    er = _prep_edges(edge_index_rev_writes, 49)

    xp = _proj(x_paper, params["lin_in"]["paper"]["w"],
               params["lin_in"]["paper"]["b"], [HID], act="relu")[0]
    xa = _proj(x_author, params["lin_in"]["author"]["w"],
               params["lin_in"]["author"]["b"], [HID], act="relu")[0]

    for layer in params["layers"]:
        pp = layer["node"]["paper"]
        pa = layer["node"]["author"]
        kwc, kbc, vwc, vbc = _fold(layer, "cites", "paper")
        kwr, kbr, vwr, vbr = _fold(layer, "rev_writes", "paper")
        kww, kbw, vww, vbw = _fold(layer, "writes", "author")

        wp = jnp.concatenate([pp["q_w"], kwc, kwr, vwc, vwr], axis=1)
        bp = jnp.concatenate([pp["q_b"], kbc, kbr, vbc, vbr])
        widths_p = [128, 128, 128] + [16] * 16
        outs_p = _proj(xp, wp, bp, widths_p)
        qp, kc, kr = outs_p[0], outs_p[1], outs_p[2]
        vc = outs_p[3:11]
        vr = outs_p[11:19]

        wa = jnp.concatenate([pa["q_w"], kww, vww], axis=1)
        ba = jnp.concatenate([pa["q_b"], kbw, vbw])
        xa_pad = jnp.concatenate(
            [xa, jnp.zeros((N_PAPER - N_AUTHOR, HID), jnp.float32)])
        outs_a = _proj(xa_pad, wa, ba, [128, 128] + [16] * 8)
        qa, kw_ = outs_a[0], outs_a[1]
        vw8 = outs_a[2:10]

        sck = _make_sc(NB_SC)
        numc, denc = sck(ec[0], qp, kc, *vc, *ec[1:])
        numw, denw = sck(ew_[0], qp, kw_, *vw8, *ew_[1:])
        numr, denr = sck(er[0], qa, kr, *vr, *er[1:])

        blp = jnp.broadcast_to(jax.nn.sigmoid(pp["skip"]), (1, HID))
        bla = jnp.broadcast_to(jax.nn.sigmoid(pa["skip"]), (1, HID))
        xp_new = _combine(
            [(numc, denc), (numw, denw)],
            xp, pp["a_w"], pp["a_b"], blp)
        xa_new = _combine(
            [(numr, denr)],
            xa, pa["a_w"], pa["a_b"], bla)
        xp, xa = xp_new, xa_new

    out = _proj(xp, params["lin_out"]["w"], params["lin_out"]["b"], [OUT])[0]
    return (out, xp, xa)


# speedup vs baseline: 9.6053x; 1.0189x over previous
import functools

import jax
import jax.numpy as jnp
import numpy as np
from jax import lax
from jax.experimental import pallas as pl
from jax.experimental.pallas import tpu as pltpu
from jax.experimental.pallas import tpu_sc as plsc

H = 8
D = 16
HID = 128
OUT = 64
NCORE = 2
NSUB = 16
NW = NCORE * NSUB
B = 64

N_PAPER = 50000
N_AUTHOR = 20000


def _proj_body(x_ref, w_ref, b_ref, *o_refs, widths, act):
    y = jnp.dot(x_ref[...], w_ref[...], preferred_element_type=jnp.float32)
    y = y + b_ref[...]
    if act == "relu":
        y = jnp.maximum(y, 0.0)
    off = 0
    for r, w in zip(o_refs, widths):
        r[...] = y[:, off:off + w]
        off += w


def _proj(x, wwide, bwide, widths, act=None, bn=400):
    n = x.shape[0]
    wt = wwide.shape[1]
    grid = (n // bn,)
    return pl.pallas_call(
        functools.partial(_proj_body, widths=tuple(widths), act=act),
        grid=grid,
        in_specs=[
            pl.BlockSpec((bn, 128), lambda i: (i, 0)),
            pl.BlockSpec((128, wt), lambda i: (0, 0)),
            pl.BlockSpec((1, wt), lambda i: (0, 0)),
        ],
        out_specs=[pl.BlockSpec((bn, w), lambda i: (i, 0)) for w in widths],
        out_shape=[jax.ShapeDtypeStruct((n, w), jnp.float32) for w in widths],
    )(x, wwide, bwide.reshape(1, wt))


def _combine_body(*refs, n_et, bn):
    num_refs = [refs[2 * e] for e in range(n_et)]
    den_refs = [refs[2 * e + 1] for e in range(n_et)]
    e8_ref, x_ref, aw_ref, ab_ref, bl_ref = refs[2 * n_et:2 * n_et + 5]
    o_ref = refs[2 * n_et + 5]
    acc = jnp.zeros((bn, HID), jnp.float32)
    for e in range(n_et):
        nr = num_refs[e][...]
        dr = den_refs[e][...]
        ntot = nr[0] + nr[1]
        dtot = dr[0] + dr[1]
        ncat = jnp.concatenate([ntot[p] for p in range(H)], axis=1)
        dx = jnp.dot(dtot, e8_ref[...], preferred_element_type=jnp.float32)
        acc = acc + ncat / (dx + 1e-16)
    g = jax.nn.gelu(acc, approximate=True)
    o = jnp.dot(g, aw_ref[...], preferred_element_type=jnp.float32) + ab_ref[...]
    bl = bl_ref[...]
    o_ref[...] = bl * o + (1.0 - bl) * x_ref[...]


def _combine(num_den_list, x_prev, aw, ab, blend_vec, bn=400):
    n = x_prev.shape[0]
    n_et = len(num_den_list)
    e8np = np.zeros((16, 128), np.float32)
    for h in range(8):
        e8np[h, 16 * h:16 * (h + 1)] = 1.0
    e8 = jnp.asarray(e8np)
    args = []
    in_specs = []
    def _nmap(i):
        return (0, 0, jnp.where(i >= RNG // bn, i + 1, i), 0)

    def _dmap(i):
        return (0, jnp.where(i >= RNG // bn, i + 1, i), 0)

    for (num, den) in num_den_list:
        args += [num, den]
        in_specs += [
            pl.BlockSpec((2, 8, bn, 16), _nmap),
            pl.BlockSpec((2, bn, 16), _dmap),
        ]
    args += [e8, x_prev, aw, ab.reshape(1, HID), blend_vec]
    in_specs += [
        pl.BlockSpec((16, 128), lambda i: (0, 0)),
        pl.BlockSpec((bn, 128), lambda i: (i, 0)),
        pl.BlockSpec((128, 128), lambda i: (0, 0)),
        pl.BlockSpec((1, 128), lambda i: (0, 0)),
        pl.BlockSpec((1, 128), lambda i: (0, 0)),
    ]
    return pl.pallas_call(
        functools.partial(_combine_body, n_et=n_et, bn=bn),
        grid=(n // bn,),
        in_specs=in_specs,
        out_specs=pl.BlockSpec((bn, 128), lambda i: (i, 0)),
        out_shape=jax.ShapeDtypeStruct((n, 128), jnp.float32),
    )(*args)


NPR = 26000
RNG = 25600


def _sc_body(meta_h, qtab, ktab, vt0, vt1, vt2, vt3, vt4, vt5, vt6, vt7,
             src_h, dstg_h, dsts_h,
             num_h, den_h,
             meta_v, src_v, dstg_v, dsts_v, didx_v, qrows, krows, vrows,
             msgv, denrows, exv, zv, num_sh, den_sh, sem, sem2, *, NB):
    c = lax.axis_index("c")
    s = lax.axis_index("s")
    wid = s * NCORE + c
    vts = [vt0, vt1, vt2, vt3, vt4, vt5, vt6, vt7]

    pltpu.sync_copy(meta_h.at[wid], meta_v)
    mvec = meta_v[pl.ds(0, 16)]
    nb_used = mvec[0]
    b0_hi = mvec[1]
    b1_lo = mvec[2]

    z16 = jnp.zeros((16,), jnp.float32)

    def _zrow(r, _):
        zv[r, pl.ds(0, 16)] = z16
        return 0
    lax.fori_loop(0, 128, _zrow, 0)

    rows_per_tile = NPR // NSUB
    nfull = rows_per_tile // 128
    nrem = rows_per_tile - nfull * 128
    r0 = s * rows_per_tile

    def _zero_rows(buf):
        def zb(i, _):
            pltpu.sync_copy(zv, buf.at[pl.ds(r0 + i * 128, 128)])
            return 0
        lax.fori_loop(0, nfull, zb, 0)
        pltpu.sync_copy(zv.at[pl.ds(0, nrem)],
                        buf.at[pl.ds(r0 + nfull * 128, nrem)])

    def _out_rows(buf, dst, roff):
        def ob(i, _):
            off = r0 + i * 128
            pltpu.sync_copy(buf.at[pl.ds(off, 128)],
                            dst.at[pl.ds(roff + off, 128)])
            return 0
        lax.fori_loop(0, nfull, ob, 0)
        off = r0 + nfull * 128
        pltpu.sync_copy(buf.at[pl.ds(off, nrem)],
                        dst.at[pl.ds(roff + off, nrem)])

    _zero_rows(num_sh)
    _zero_rows(den_sh)

    def ex_batch(j, _):
        pltpu.sync_copy(dstg_h.at[wid, j], dstg_v)
        pltpu.sync_copy(src_h.at[wid, j], src_v)
        hq = pltpu.async_copy(qtab.at[dstg_v], qrows, sem)
        hk = pltpu.async_copy(ktab.at[src_v], krows, sem2)
        hq.wait()
        hk.wait()

        def grp(g, _):
            ev = g * 16 + lax.iota(jnp.int32, 16)
            for h in range(H):
                acc = jnp.zeros((16,), jnp.float32)
                for d in range(D):
                    col = jnp.full((16,), h * D + d, jnp.int32)
                    acc = acc + (plsc.load_gather(qrows, [ev, col]) *
                                 plsc.load_gather(krows, [ev, col]))
                exv[h, pl.ds(j * B + g * 16, 16)] = jnp.exp(acc)
            return 0
        lax.fori_loop(0, B // 16, grp, 0)
        return 0
    lax.fori_loop(0, nb_used, ex_batch, 0)

    for r in range(2):
        base = r * RNG
        j_lo = 0 if r == 0 else b1_lo
        j_hi = b0_hi if r == 0 else nb_used

        def db(j, _):
            pltpu.sync_copy(dsts_h.at[wid, j], dsts_v)

            def dg(g, _):
                dv = dsts_v[pl.ds(g * 16, 16)]
                ld = dv - base
                ok = (ld >= 0) & (ld < RNG)
                didx_v[j, pl.ds(g * 16, 16)] = jnp.where(ok, ld, RNG)
                return 0
            lax.fori_loop(0, B // 16, dg, 0)
            return 0
        lax.fori_loop(j_lo, j_hi, db, 0)

        plsc.subcore_barrier()

        def den_batch(j, _):
            def dgrp(g, _):
                hmod = lax.iota(jnp.int32, 16) & 7
                for i in range(16):
                    e = g * 16 + i
                    ge = jnp.full((16,), j * B + e, jnp.int32)
                    denrows[e, pl.ds(0, 16)] = plsc.load_gather(exv, [hmod, ge])
                return 0
            lax.fori_loop(0, B // 16, dgrp, 0)
            pltpu.sync_copy(denrows, den_sh.at[didx_v.at[j]], add=True)
            return 0
        lax.fori_loop(j_lo, j_hi, den_batch, 0)

        for p in range(H):
            if p > 0:
                plsc.subcore_barrier()

            def cp_batch(j, _):
                pltpu.sync_copy(src_h.at[wid, j], src_v)
                pltpu.async_copy(vts[p].at[src_v], vrows, sem).wait()

                def grp(g, _):
                    e0 = g * 16
                    exvec = exv[p, pl.ds(j * B + e0, 16)]
                    for i in range(16):
                        msgv[e0 + i, pl.ds(0, 16)] = (
                            vrows[e0 + i, pl.ds(0, 16)] * exvec[i])
                    return 0
                lax.fori_loop(0, B // 16, grp, 0)
                pltpu.sync_copy(msgv, num_sh.at[didx_v.at[j]], add=True)
                return 0
            lax.fori_loop(j_lo, j_hi, cp_batch, 0)

            plsc.subcore_barrier()

            if p == 0:
                _out_rows(den_sh, den_h.at[c], r * NPR)
                if r == 0:
                    _zero_rows(den_sh)
            _out_rows(num_sh, num_h.at[c, p], r * NPR)
            if not (r == 1 and p == H - 1):
                _zero_rows(num_sh)


OUT_ROWS = 2 * NPR


@functools.lru_cache(maxsize=None)
def _make_sc(NB):
    ew = NB * B
    mesh = plsc.VectorSubcoreMesh(core_axis_name="c", subcore_axis_name="s")
    return functools.partial(
        pl.kernel,
        mesh=mesh,
        compiler_params=pltpu.CompilerParams(
            needs_layout_passes=False, use_tc_tiling_on_sc=False),
        out_type=[
            jax.ShapeDtypeStruct((NCORE, H, OUT_ROWS, 16), jnp.float32),
            jax.ShapeDtypeStruct((NCORE, OUT_ROWS, 16), jnp.float32),
        ],
        scratch_types=[
            pltpu.VMEM((16,), jnp.int32),
            pltpu.VMEM((B,), jnp.int32),
            pltpu.VMEM((B,), jnp.int32),
            pltpu.VMEM((B,), jnp.int32),
            pltpu.VMEM((NB, B), jnp.int32),
            pltpu.VMEM((B, 128), jnp.float32),
            pltpu.VMEM((B, 128), jnp.float32),
            pltpu.VMEM((B, 16), jnp.float32),
            pltpu.VMEM((B, 16), jnp.float32),
            pltpu.VMEM((B, 16), jnp.float32),
            pltpu.VMEM((H, ew), jnp.float32),
            pltpu.VMEM((128, 16), jnp.float32),
            pltpu.VMEM_SHARED((NPR, 16), jnp.float32),
            pltpu.VMEM_SHARED((NPR, 16), jnp.float32),
            pltpu.SemaphoreType.DMA,
            pltpu.SemaphoreType.DMA,
        ],
    )(functools.partial(_sc_body, NB=NB))


NB_SC = 98
GARBAGE_ROW = 50000


def _prep_edges(ei, nb_real):
    e = ei.shape[1]
    order = jnp.argsort(ei[1] >= RNG, stable=True)
    s_srt = ei[0][order]
    d_srt = ei[1][order]
    ewr = nb_real * B
    epad = NW * ewr
    src = jnp.concatenate([s_srt, jnp.zeros((epad - e,), jnp.int32)])
    dstg = jnp.concatenate([d_srt, jnp.zeros((epad - e,), jnp.int32)])
    dsts = jnp.concatenate([d_srt, jnp.full((epad - e,), GARBAGE_ROW,
                                            jnp.int32)])

    def expand(a, fill):
        a2 = a.reshape(NW, ewr)
        pad = jnp.full((NW, (NB_SC - nb_real) * B), fill, jnp.int32)
        return jnp.concatenate([a2, pad], axis=1).reshape(NW, NB_SC, B)

    count0 = jnp.sum(ei[1] < RNG).astype(jnp.int32)
    c0w = jnp.clip(count0 - jnp.arange(NW, dtype=jnp.int32) * ewr, 0, ewr)
    b0_hi = (c0w + B - 1) // B
    b1_lo = c0w // B
    meta = jnp.stack(
        [jnp.full((NW,), nb_real, jnp.int32), b0_hi, b1_lo]
        + [jnp.zeros((NW,), jnp.int32)] * 13, axis=1)
    return (meta, expand(src, 0), expand(dstg, 0),
            expand(dsts, GARBAGE_ROW))


def _block_diag(a):
    out = jnp.zeros((H * D, H * D), jnp.float32)
    for h in range(H):
        out = out.at[h * D:(h + 1) * D, h * D:(h + 1) * D].set(a[h])
    return out


def _fold(layer, et, src_t):
    pe = layer["edge"][et]
    p = layer["node"][src_t]
    scale = jnp.repeat(pe["p_rel"] / jnp.sqrt(jnp.float32(D)), D)
    bda = _block_diag(pe["a_rel"])
    bdm = _block_diag(pe["m_rel"])
    kw = (p["k_w"] @ bda) * scale[None, :]
    kb = (p["k_b"] @ bda) * scale
    vw = p["v_w"] @ bdm
    vb = p["v_b"] @ bdm
    return kw, kb, vw, vb


def kernel(x_paper, x_author, edge_index_cites, edge_index_writes,
           edge_index_rev_writes, params):
    ec = _prep_edges(edge_index_cites, 98)
    ew_ = _prep_edges(edge_index_writes, 49)
    er = _prep_edges(edge_index_rev_writes, 49)

    xp = _proj(x_paper, params["lin_in"]["paper"]["w"],
               params["lin_in"]["paper"]["b"], [HID], act="relu")[0]
    xa = _proj(x_author, params["lin_in"]["author"]["w"],
               params["lin_in"]["author"]["b"], [HID], act="relu")[0]

    for layer in params["layers"]:
        pp = layer["node"]["paper"]
        pa = layer["node"]["author"]
        kwc, kbc, vwc, vbc = _fold(layer, "cites", "paper")
        kwr, kbr, vwr, vbr = _fold(layer, "rev_writes", "paper")
        kww, kbw, vww, vbw = _fold(layer, "writes", "author")

        wp = jnp.concatenate([pp["q_w"], kwc, kwr, vwc, vwr], axis=1)
        bp = jnp.concatenate([pp["q_b"], kbc, kbr, vbc, vbr])
        widths_p = [128, 128, 128] + [16] * 16
        outs_p = _proj(xp, wp, bp, widths_p)
        qp, kc, kr = outs_p[0], outs_p[1], outs_p[2]
        vc = outs_p[3:11]
        vr = outs_p[11:19]

        wa = jnp.concatenate([pa["q_w"], kww, vww], axis=1)
        ba = jnp.concatenate([pa["q_b"], kbw, vbw])
        xa_pad = jnp.concatenate(
            [xa, jnp.zeros((N_PAPER - N_AUTHOR, HID), jnp.float32)])
        outs_a = _proj(xa_pad, wa, ba, [128, 128] + [16] * 8)
        qa, kw_ = outs_a[0], outs_a[1]
        vw8 = outs_a[2:10]

        sck = _make_sc(NB_SC)
        numc, denc = sck(ec[0], qp, kc, *vc, *ec[1:])
        numw, denw = sck(ew_[0], qp, kw_, *vw8, *ew_[1:])
        numr, denr = sck(er[0], qa, kr, *vr, *er[1:])

        blp = jnp.broadcast_to(jax.nn.sigmoid(pp["skip"]), (1, HID))
        bla = jnp.broadcast_to(jax.nn.sigmoid(pa["skip"]), (1, HID))
        xp_new = _combine(
            [(numc, denc), (numw, denw)],
            xp, pp["a_w"], pp["a_b"], blp)
        xa_new = _combine(
            [(numr, denr)],
            xa, pa["a_w"], pa["a_b"], bla)
        xp, xa = xp_new, xa_new

    out = _proj(xp, params["lin_out"]["w"], params["lin_out"]["b"], [OUT])[0]
    return (out, xp, xa)
